# baseline (speedup 1.0000x reference)
import functools

import jax
import jax.numpy as jnp
from jax import lax
from jax.experimental import pallas as pl
from jax.experimental.pallas import tpu as pltpu
from jax.experimental.pallas import tpu_sc as plsc

TAU = 1.0
BLK = 256
NW = 32
GCH = 64
CCH = 32


def _ffn_body(be_ref, xg_ref, w1_ref, b1_ref, w2_ref, b2_ref, wv_ref, out_ref):
    del be_ref
    xb = xg_ref[...]
    h = jnp.maximum(
        lax.dot_general(xb, w1_ref[0], (((1,), (1,)), ((), ())),
                        preferred_element_type=jnp.float32) + b1_ref[0], 0.0)
    o = lax.dot_general(h, w2_ref[0], (((1,), (1,)), ((), ())),
                        preferred_element_type=jnp.float32) + b2_ref[0]
    out_ref[...] = o * wv_ref[...]


def _ffn_call(block_expert, xg, We1, be1, We2, be2, wvec2d):
    P_cap, D = xg.shape
    E, H, _ = We1.shape
    NB = P_cap // BLK
    spec = pltpu.PrefetchScalarGridSpec(
        num_scalar_prefetch=1,
        grid=(NB,),
        in_specs=[
            pl.BlockSpec((BLK, D), lambda b, be: (b, 0)),
            pl.BlockSpec((1, H, D), lambda b, be: (be[b], 0, 0)),
            pl.BlockSpec((1, 1, H), lambda b, be: (be[b], 0, 0)),
            pl.BlockSpec((1, D, H), lambda b, be: (be[b], 0, 0)),
            pl.BlockSpec((1, 1, D), lambda b, be: (be[b], 0, 0)),
            pl.BlockSpec((BLK, 1), lambda b, be: (b, 0)),
        ],
        out_specs=pl.BlockSpec((BLK, D), lambda b, be: (b, 0)),
    )
    return pl.pallas_call(
        _ffn_body,
        grid_spec=spec,
        out_shape=jax.ShapeDtypeStruct((P_cap, D), jnp.float32),
    )(block_expert, xg, We1, be1.reshape(E, 1, H), We2,
      be2.reshape(E, 1, D), wvec2d)


def _sc_gather(x_flat, idx):
    P_cap = idx.shape[0]
    D = x_flat.shape[1]
    per = P_cap // NW
    nch = per // GCH
    mesh = plsc.VectorSubcoreMesh(core_axis_name="c", subcore_axis_name="s")

    @functools.partial(
        pl.kernel,
        out_type=jax.ShapeDtypeStruct((P_cap, D), jnp.float32),
        mesh=mesh,
        scratch_types=[
            pltpu.VMEM((GCH,), jnp.int32),
            pltpu.VMEM((GCH, D), jnp.float32),
            pltpu.SemaphoreType.DMA,
        ],
    )
    def k(x_hbm, idx_hbm, out_hbm, idx_v, rows_v, sem):
        wid = lax.axis_index("s") * 2 + lax.axis_index("c")
        base = wid * per
        for c in range(nch):
            off = base + c * GCH
            pltpu.sync_copy(idx_hbm.at[pl.ds(off, GCH)], idx_v)
            pltpu.async_copy(x_hbm.at[idx_v], rows_v, sem).wait()
            pltpu.sync_copy(rows_v, out_hbm.at[pl.ds(off, GCH)])

    return k(x_flat, idx)


def _sc_combine(y, slot0, slot1):
    T = slot0.shape[0]
    D = y.shape[1]
    per = T // NW
    nch = per // CCH
    mesh = plsc.VectorSubcoreMesh(core_axis_name="c", subcore_axis_name="s")

    @functools.partial(
        pl.kernel,
        out_type=jax.ShapeDtypeStruct((T, D), jnp.float32),
        mesh=mesh,
        scratch_types=[
            pltpu.VMEM((CCH,), jnp.int32),
            pltpu.VMEM((CCH,), jnp.int32),
            pltpu.VMEM((CCH, D), jnp.float32),
            pltpu.VMEM((CCH, D), jnp.float32),
            pltpu.SemaphoreType.DMA,
            pltpu.SemaphoreType.DMA,
        ],
    )
    def k(y_hbm, s0_hbm, s1_hbm, out_hbm, i0_v, i1_v, a_v, b_v, sem0, sem1):
        wid = lax.axis_index("s") * 2 + lax.axis_index("c")
        base = wid * per
        for c in range(nch):
            off = base + c * CCH
            pltpu.sync_copy(s0_hbm.at[pl.ds(off, CCH)], i0_v)
            pltpu.sync_copy(s1_hbm.at[pl.ds(off, CCH)], i1_v)
            cp0 = pltpu.async_copy(y_hbm.at[i0_v], a_v, sem0)
            cp1 = pltpu.async_copy(y_hbm.at[i1_v], b_v, sem1)
            cp0.wait()
            cp1.wait()

            def row_add(i, carry):
                for j in range(D // 16):
                    sl = pl.ds(j * 16, 16)
                    a_v[i, sl] = a_v[i, sl] + b_v[i, sl]
                return carry

            lax.fori_loop(0, CCH, row_add, 0)
            pltpu.sync_copy(a_v, out_hbm.at[pl.ds(off, CCH)])

    return k(y, slot0, slot1)


def _route(topk_idx, topk_w, T, E):
    P = T * 2
    P_cap = P + E * BLK
    NB = P_cap // BLK
    eidx = topk_idx.reshape(P)
    wflat = topk_w.reshape(P)
    onehot = (eidx[:, None] == jnp.arange(E, dtype=jnp.int32)[None, :]
              ).astype(jnp.int32)
    cum = jnp.cumsum(onehot, axis=0)
    rank = jnp.take_along_axis(cum, eidx[:, None], axis=1)[:, 0] - 1
    counts = cum[-1]
    padded = ((counts + BLK - 1) // BLK) * BLK
    ends = jnp.cumsum(padded)
    poff = ends - padded
    s = poff[eidx] + rank
    src_token = jnp.zeros((P_cap,), jnp.int32).at[s].set(
        jnp.arange(P, dtype=jnp.int32) // 2)
    wvec = jnp.zeros((P_cap,), jnp.float32).at[s].set(wflat)
    bstart = jnp.arange(NB, dtype=jnp.int32) * BLK
    block_expert = jnp.minimum(
        jnp.searchsorted(ends, bstart, side="right").astype(jnp.int32), E - 1)
    slot = s.reshape(T, 2)
    return src_token, wvec.reshape(P_cap, 1), block_expert, slot[:, 0], slot[:, 1]


def kernel(x, Wg1, bg1, Wg2, bg2, We1, be1, We2, be2, snr):
    Bc, Lc, D = x.shape
    T = Bc * Lc
    E = Wg2.shape[0]
    x_flat = x.reshape(T, D)

    snr_col = jnp.full((T, 1), jnp.float32(snr))
    gate_in = jnp.concatenate([x_flat, snr_col], axis=-1)
    hg = jax.nn.relu(gate_in @ Wg1.T + bg1)
    logits = hg @ Wg2.T + bg2
    u = jax.random.uniform(jax.random.key(42), logits.shape, jnp.float32,
                           1e-6, 1.0 - 1e-6)
    gumbel = -jnp.log(-jnp.log(u + 1e-9) + 1e-9)
    gate_scores = jax.nn.softmax((logits + gumbel) / TAU, axis=-1)
    topk_scores, topk_idx = jax.lax.top_k(gate_scores, 2)
    onehots = jax.nn.one_hot(topk_idx, E, dtype=x.dtype)
    expert_mask = jnp.clip(jnp.sum(onehots, axis=1), 0.0, 1.0)

    src_token, wvec2d, block_expert, slot0, slot1 = _route(
        topk_idx.astype(jnp.int32), topk_scores, T, E)
    xg = _sc_gather(x_flat, src_token)
    y = _ffn_call(block_expert, xg, We1, be1, We2, be2, wvec2d)
    out_flat = _sc_combine(y, slot0, slot1)
    return out_flat.reshape(Bc, Lc, D), gate_scores, expert_mask


# speedup vs baseline: 1.0140x; 1.0140x over previous
import functools

import jax
import jax.numpy as jnp
from jax import lax
from jax.experimental import pallas as pl
from jax.experimental.pallas import tpu as pltpu
from jax.experimental.pallas import tpu_sc as plsc

TAU = 1.0
BLK = 256
NW = 32
GCH = 40
CCH = 16


def _ffn_body(be_ref, xg_ref, w1_ref, b1_ref, w2_ref, b2_ref, wv_ref, out_ref):
    del be_ref
    xb = xg_ref[...]
    h = jnp.maximum(
        lax.dot_general(xb, w1_ref[0], (((1,), (1,)), ((), ())),
                        preferred_element_type=jnp.float32) + b1_ref[0], 0.0)
    o = lax.dot_general(h, w2_ref[0], (((1,), (1,)), ((), ())),
                        preferred_element_type=jnp.float32) + b2_ref[0]
    out_ref[...] = o * wv_ref[...]


def _ffn_call(block_expert, xg, We1, be1, We2, be2, wvec2d):
    P_cap, D = xg.shape
    E, H, _ = We1.shape
    NB = P_cap // BLK
    spec = pltpu.PrefetchScalarGridSpec(
        num_scalar_prefetch=1,
        grid=(NB,),
        in_specs=[
            pl.BlockSpec((BLK, D), lambda b, be: (b, 0)),
            pl.BlockSpec((1, H, D), lambda b, be: (be[b], 0, 0)),
            pl.BlockSpec((1, 1, H), lambda b, be: (be[b], 0, 0)),
            pl.BlockSpec((1, D, H), lambda b, be: (be[b], 0, 0)),
            pl.BlockSpec((1, 1, D), lambda b, be: (be[b], 0, 0)),
            pl.BlockSpec((BLK, 1), lambda b, be: (b, 0)),
        ],
        out_specs=pl.BlockSpec((BLK, D), lambda b, be: (b, 0)),
    )
    return pl.pallas_call(
        _ffn_body,
        grid_spec=spec,
        out_shape=jax.ShapeDtypeStruct((P_cap, D), jnp.float32),
    )(block_expert, xg, We1, be1.reshape(E, 1, H), We2,
      be2.reshape(E, 1, D), wvec2d)


def _sc_gather(x_flat, idx):
    P_cap = idx.shape[0]
    D = x_flat.shape[1]
    per = P_cap // NW
    nch = per // GCH
    mesh = plsc.VectorSubcoreMesh(core_axis_name="c", subcore_axis_name="s")

    @functools.partial(
        pl.kernel,
        out_type=jax.ShapeDtypeStruct((P_cap, D), jnp.float32),
        mesh=mesh,
        scratch_types=[
            pltpu.VMEM((2, GCH), jnp.int32),
            pltpu.VMEM((2, GCH, D), jnp.float32),
            pltpu.SemaphoreType.DMA,
            pltpu.SemaphoreType.DMA,
        ],
    )
    def k(x_hbm, idx_hbm, out_hbm, idx_v, rows_v, sem0, sem1):
        wid = lax.axis_index("s") * 2 + lax.axis_index("c")
        base = wid * per
        sems = (sem0, sem1)

        def start(c):
            sl = c % 2
            off = base + c * GCH
            pltpu.sync_copy(idx_hbm.at[pl.ds(off, GCH)], idx_v.at[sl])
            return pltpu.async_copy(x_hbm.at[idx_v.at[sl]], rows_v.at[sl],
                                    sems[sl])

        cps = [None, None]
        cps[0] = start(0)
        for c in range(nch):
            sl = c % 2
            if c + 1 < nch:
                cps[(c + 1) % 2] = start(c + 1)
            cps[sl].wait()
            pltpu.sync_copy(rows_v.at[sl], out_hbm.at[pl.ds(base + c * GCH, GCH)])

    return k(x_flat, idx)


def _sc_combine(y, slot0, slot1):
    T = slot0.shape[0]
    D = y.shape[1]
    per = T // NW
    nch = per // CCH
    mesh = plsc.VectorSubcoreMesh(core_axis_name="c", subcore_axis_name="s")

    @functools.partial(
        pl.kernel,
        out_type=jax.ShapeDtypeStruct((T, D), jnp.float32),
        mesh=mesh,
        scratch_types=[
            pltpu.VMEM((2, CCH), jnp.int32),
            pltpu.VMEM((2, CCH), jnp.int32),
            pltpu.VMEM((2, CCH, D), jnp.float32),
            pltpu.VMEM((2, CCH, D), jnp.float32),
            pltpu.SemaphoreType.DMA,
            pltpu.SemaphoreType.DMA,
            pltpu.SemaphoreType.DMA,
            pltpu.SemaphoreType.DMA,
        ],
    )
    def k(y_hbm, s0_hbm, s1_hbm, out_hbm, i0_v, i1_v, a_v, b_v,
          sa0, sa1, sb0, sb1):
        wid = lax.axis_index("s") * 2 + lax.axis_index("c")
        base = wid * per
        sas = (sa0, sa1)
        sbs = (sb0, sb1)

        def start(c):
            sl = c % 2
            off = base + c * CCH
            pltpu.sync_copy(s0_hbm.at[pl.ds(off, CCH)], i0_v.at[sl])
            pltpu.sync_copy(s1_hbm.at[pl.ds(off, CCH)], i1_v.at[sl])
            return (pltpu.async_copy(y_hbm.at[i0_v.at[sl]], a_v.at[sl], sas[sl]),
                    pltpu.async_copy(y_hbm.at[i1_v.at[sl]], b_v.at[sl], sbs[sl]))

        cps = [None, None]
        cps[0] = start(0)
        for c in range(nch):
            sl = c % 2
            if c + 1 < nch:
                cps[(c + 1) % 2] = start(c + 1)
            cps[sl][0].wait()
            cps[sl][1].wait()

            def row_add(i, carry):
                for j in range(D // 16):
                    s_ = pl.ds(j * 16, 16)
                    plsc.addupdate(a_v.at[sl, i, s_], b_v[sl, i, s_])
                return carry

            lax.fori_loop(0, CCH, row_add, 0)
            pltpu.sync_copy(a_v.at[sl], out_hbm.at[pl.ds(base + c * CCH, CCH)])

    return k(y, slot0, slot1)


def _route(topk_idx, topk_w, T, E):
    P = T * 2
    P_cap = P + E * BLK
    NB = P_cap // BLK
    eidx = topk_idx.reshape(P)
    wflat = topk_w.reshape(P)
    onehot = (eidx[:, None] == jnp.arange(E, dtype=jnp.int32)[None, :]
              ).astype(jnp.int32)
    cum = jnp.cumsum(onehot, axis=0)
    rank = jnp.take_along_axis(cum, eidx[:, None], axis=1)[:, 0] - 1
    counts = cum[-1]
    padded = ((counts + BLK - 1) // BLK) * BLK
    ends = jnp.cumsum(padded)
    poff = ends - padded
    s = poff[eidx] + rank
    src_token = jnp.zeros((P_cap,), jnp.int32).at[s].set(
        jnp.arange(P, dtype=jnp.int32) // 2)
    wvec = jnp.zeros((P_cap,), jnp.float32).at[s].set(wflat)
    bstart = jnp.arange(NB, dtype=jnp.int32) * BLK
    block_expert = jnp.minimum(
        jnp.searchsorted(ends, bstart, side="right").astype(jnp.int32), E - 1)
    slot = s.reshape(T, 2)
    return src_token, wvec.reshape(P_cap, 1), block_expert, slot[:, 0], slot[:, 1]


def kernel(x, Wg1, bg1, Wg2, bg2, We1, be1, We2, be2, snr):
    Bc, Lc, D = x.shape
    T = Bc * Lc
    E = Wg2.shape[0]
    x_flat = x.reshape(T, D)

    snr_col = jnp.full((T, 1), jnp.float32(snr))
    gate_in = jnp.concatenate([x_flat, snr_col], axis=-1)
    hg = jax.nn.relu(gate_in @ Wg1.T + bg1)
    logits = hg @ Wg2.T + bg2
    u = jax.random.uniform(jax.random.key(42), logits.shape, jnp.float32,
                           1e-6, 1.0 - 1e-6)
    gumbel = -jnp.log(-jnp.log(u + 1e-9) + 1e-9)
    gate_scores = jax.nn.softmax((logits + gumbel) / TAU, axis=-1)
    topk_scores, topk_idx = jax.lax.top_k(gate_scores, 2)
    onehots = jax.nn.one_hot(topk_idx, E, dtype=x.dtype)
    expert_mask = jnp.clip(jnp.sum(onehots, axis=1), 0.0, 1.0)

    src_token, wvec2d, block_expert, slot0, slot1 = _route(
        topk_idx.astype(jnp.int32), topk_scores, T, E)
    xg = _sc_gather(x_flat, src_token)
    y = _ffn_call(block_expert, xg, We1, be1, We2, be2, wvec2d)
    out_flat = _sc_combine(y, slot0, slot1)
    return out_flat.reshape(Bc, Lc, D), gate_scores, expert_mask
